# baseline (speedup 1.0000x reference)
import functools

import jax
import jax.numpy as jnp
from jax.experimental import pallas as pl
from jax.experimental.pallas import tpu as pltpu

_Q0, _T0, _E0, _C0, _B0 = 0, 8, 16, 48, 56
_HB = 64
_NB = _HB * 128


def _head_kernel(f_ref, w_ref, b_ref, quat_ref, trans_ref, embed_ref,
                 conf_ref, bbox_ref, *r_refs):
    f = f_ref[0]
    out = jax.lax.dot_general(
        w_ref[...], f, (((1,), (0,)), ((), ())),
        preferred_element_type=jnp.float32)
    out = out + b_ref[...].reshape(64, 1, 1)

    q4 = out[_Q0:_Q0 + 4]
    trans_ref[0] = out[_T0:_T0 + 3]
    embed_ref[0] = out[_E0:_E0 + 32]
    conf_ref[0] = jax.nn.sigmoid(out[_C0:_C0 + 1])
    bbox_ref[0] = out[_B0:_B0 + 4]

    s = jnp.sum(q4 * q4, axis=0, keepdims=True)
    inv = 1.0 / (jnp.sqrt(s) + 1e-8)
    qn = q4 * inv
    quat_ref[0] = qn

    w = qn[0]
    x = qn[1]
    y = qn[2]
    z = qn[3]
    xx = 2.0 * x * x
    yy = 2.0 * y * y
    zz = 2.0 * z * z
    xy = 2.0 * x * y
    xz = 2.0 * x * z
    yz = 2.0 * y * z
    xw = 2.0 * x * w
    yw = 2.0 * y * w
    zw = 2.0 * z * w
    rs = (1.0 - (yy + zz), xy - zw, xz + yw,
          xy + zw, 1.0 - (xx + zz), yz - xw,
          xz - yw, yz + xw, 1.0 - (xx + yy))
    for r_ref, r in zip(r_refs, rs):
        r_ref[0] = r


@functools.partial(jax.jit, static_argnames=())
def kernel(feat_map, W_quat, b_quat, W_trans, b_trans, W_embed, b_embed,
           W_conf, b_conf, W_bbox, b_bbox):
    B, C, H, W = feat_map.shape

    wall = jnp.zeros((64, C), jnp.float32)
    wall = wall.at[_Q0:_Q0 + 4].set(W_quat)
    wall = wall.at[_T0:_T0 + 3].set(W_trans)
    wall = wall.at[_E0:_E0 + 32].set(W_embed)
    wall = wall.at[_C0:_C0 + 1].set(W_conf)
    wall = wall.at[_B0:_B0 + 4].set(W_bbox)
    ball = jnp.zeros((64, 1), jnp.float32)
    ball = ball.at[_Q0:_Q0 + 4, 0].set(b_quat)
    ball = ball.at[_T0:_T0 + 3, 0].set(b_trans)
    ball = ball.at[_E0:_E0 + 32, 0].set(b_embed)
    ball = ball.at[_C0:_C0 + 1, 0].set(b_conf)
    ball = ball.at[_B0:_B0 + 4, 0].set(b_bbox)

    grid = (B, H // _HB)
    ch_spec = lambda o: pl.BlockSpec((1, o, _HB, 128), lambda b, i: (b, 0, i, 0))
    pl_spec = pl.BlockSpec((1, _HB, 128), lambda b, i: (b, i, 0))
    outs = pl.pallas_call(
        _head_kernel,
        grid=grid,
        in_specs=[
            ch_spec(C),
            pl.BlockSpec((64, C), lambda b, i: (0, 0)),
            pl.BlockSpec((64, 1), lambda b, i: (0, 0)),
        ],
        out_specs=[ch_spec(4), ch_spec(3), ch_spec(32), ch_spec(1), ch_spec(4)]
                  + [pl_spec] * 9,
        out_shape=[
            jax.ShapeDtypeStruct((B, 4, H, W), jnp.float32),
            jax.ShapeDtypeStruct((B, 3, H, W), jnp.float32),
            jax.ShapeDtypeStruct((B, 32, H, W), jnp.float32),
            jax.ShapeDtypeStruct((B, 1, H, W), jnp.float32),
            jax.ShapeDtypeStruct((B, 4, H, W), jnp.float32),
        ] + [jax.ShapeDtypeStruct((B, H, W), jnp.float32)] * 9,
        compiler_params=pltpu.CompilerParams(
            dimension_semantics=("parallel", "parallel")),
    )(feat_map, wall, ball)

    quat, trans, embed, conf, bbox = outs[:5]
    r00, r01, r02, r10, r11, r12, r20, r21, r22 = outs[5:]
    row0 = jnp.stack([r00, r01, r02], axis=-1)
    row1 = jnp.stack([r10, r11, r12], axis=-1)
    row2 = jnp.stack([r20, r21, r22], axis=-1)
    global_R = jnp.stack([row0, row1, row2], axis=-2)
    return (quat, trans, embed, conf, bbox, global_R, trans)


# speedup vs baseline: 1.2189x; 1.2189x over previous
import functools

import jax
import jax.numpy as jnp
from jax.experimental import pallas as pl
from jax.experimental.pallas import tpu as pltpu

_Q0, _T0, _E0, _C0, _B0 = 0, 8, 16, 48, 56
_HB = 128
_NB = _HB * 128


def _head_kernel(f_ref, w_ref, b_ref, quat_ref, trans_ref, embed_ref,
                 conf_ref, bbox_ref, r9_ref):
    f = f_ref[0]
    out = jax.lax.dot_general(
        w_ref[...], f, (((1,), (0,)), ((), ())),
        preferred_element_type=jnp.float32)
    out = out + b_ref[...].reshape(64, 1, 1)

    q4 = out[_Q0:_Q0 + 4]
    trans_ref[0] = out[_T0:_T0 + 3]
    embed_ref[0] = out[_E0:_E0 + 32]
    conf_ref[0] = jax.nn.sigmoid(out[_C0:_C0 + 1])
    bbox_ref[0] = out[_B0:_B0 + 4]

    s = jnp.sum(q4 * q4, axis=0, keepdims=True)
    inv = 1.0 / (jnp.sqrt(s) + 1e-8)
    qn = q4 * inv
    quat_ref[0] = qn

    w = qn[0]
    x = qn[1]
    y = qn[2]
    z = qn[3]
    xx = 2.0 * x * x
    yy = 2.0 * y * y
    zz = 2.0 * z * z
    xy = 2.0 * x * y
    xz = 2.0 * x * z
    yz = 2.0 * y * z
    xw = 2.0 * x * w
    yw = 2.0 * y * w
    zw = 2.0 * z * w
    rs = (1.0 - (yy + zz), xy - zw, xz + yw,
          xy + zw, 1.0 - (xx + zz), yz - xw,
          xz - yw, yz + xw, 1.0 - (xx + yy))
    for k, r in enumerate(rs):
        r9_ref[0, k] = r


@functools.partial(jax.jit, static_argnames=())
def kernel(feat_map, W_quat, b_quat, W_trans, b_trans, W_embed, b_embed,
           W_conf, b_conf, W_bbox, b_bbox):
    B, C, H, W = feat_map.shape

    wall = jnp.zeros((64, C), jnp.float32)
    wall = wall.at[_Q0:_Q0 + 4].set(W_quat)
    wall = wall.at[_T0:_T0 + 3].set(W_trans)
    wall = wall.at[_E0:_E0 + 32].set(W_embed)
    wall = wall.at[_C0:_C0 + 1].set(W_conf)
    wall = wall.at[_B0:_B0 + 4].set(W_bbox)
    ball = jnp.zeros((64, 1), jnp.float32)
    ball = ball.at[_Q0:_Q0 + 4, 0].set(b_quat)
    ball = ball.at[_T0:_T0 + 3, 0].set(b_trans)
    ball = ball.at[_E0:_E0 + 32, 0].set(b_embed)
    ball = ball.at[_C0:_C0 + 1, 0].set(b_conf)
    ball = ball.at[_B0:_B0 + 4, 0].set(b_bbox)

    grid = (B, H // _HB)
    ch_spec = lambda o: pl.BlockSpec((1, o, _HB, 128), lambda b, i: (b, 0, i, 0))
    pl_spec = pl.BlockSpec((1, _HB, 128), lambda b, i: (b, i, 0))
    outs = pl.pallas_call(
        _head_kernel,
        grid=grid,
        in_specs=[
            ch_spec(C),
            pl.BlockSpec((64, C), lambda b, i: (0, 0)),
            pl.BlockSpec((64, 1), lambda b, i: (0, 0)),
        ],
        out_specs=[ch_spec(4), ch_spec(3), ch_spec(32), ch_spec(1), ch_spec(4),
                   ch_spec(9)],
        out_shape=[
            jax.ShapeDtypeStruct((B, 4, H, W), jnp.float32),
            jax.ShapeDtypeStruct((B, 3, H, W), jnp.float32),
            jax.ShapeDtypeStruct((B, 32, H, W), jnp.float32),
            jax.ShapeDtypeStruct((B, 1, H, W), jnp.float32),
            jax.ShapeDtypeStruct((B, 4, H, W), jnp.float32),
            jax.ShapeDtypeStruct((B, 9, H, W), jnp.float32),
        ],
        compiler_params=pltpu.CompilerParams(
            dimension_semantics=("parallel", "parallel")),
    )(feat_map, wall, ball)

    quat, trans, embed, conf, bbox, r9 = outs
    global_R = jnp.transpose(r9, (0, 2, 3, 1)).reshape(B, H, W, 3, 3)
    return (quat, trans, embed, conf, bbox, global_R, trans)


# speedup vs baseline: 1.2303x; 1.0094x over previous
import functools

import jax
import jax.numpy as jnp
from jax.experimental import pallas as pl
from jax.experimental.pallas import tpu as pltpu

_Q0, _T0, _E0, _C0, _B0 = 0, 8, 16, 48, 56
_HB = 128
_NB = _HB * 128
_BB = 2


def _head_kernel(f_ref, w_ref, b_ref, quat_ref, trans_ref, embed_ref,
                 conf_ref, bbox_ref, r9_ref):
    for bb in range(_BB):
        f = f_ref[bb]
        out = jax.lax.dot_general(
            w_ref[...], f, (((1,), (0,)), ((), ())),
            preferred_element_type=jnp.float32)
        out = out + b_ref[...].reshape(64, 1, 1)

        q4 = out[_Q0:_Q0 + 4]
        trans_ref[bb] = out[_T0:_T0 + 3]
        embed_ref[bb] = out[_E0:_E0 + 32]
        conf_ref[bb] = jax.nn.sigmoid(out[_C0:_C0 + 1])
        bbox_ref[bb] = out[_B0:_B0 + 4]

        s = jnp.sum(q4 * q4, axis=0, keepdims=True)
        inv = 1.0 / (jnp.sqrt(s) + 1e-8)
        qn = q4 * inv
        quat_ref[bb] = qn

        w = qn[0]
        x = qn[1]
        y = qn[2]
        z = qn[3]
        xx = 2.0 * x * x
        yy = 2.0 * y * y
        zz = 2.0 * z * z
        xy = 2.0 * x * y
        xz = 2.0 * x * z
        yz = 2.0 * y * z
        xw = 2.0 * x * w
        yw = 2.0 * y * w
        zw = 2.0 * z * w
        rs = (1.0 - (yy + zz), xy - zw, xz + yw,
              xy + zw, 1.0 - (xx + zz), yz - xw,
              xz - yw, yz + xw, 1.0 - (xx + yy))
        for k, r in enumerate(rs):
            r9_ref[bb, k] = r


@functools.partial(jax.jit, static_argnames=())
def kernel(feat_map, W_quat, b_quat, W_trans, b_trans, W_embed, b_embed,
           W_conf, b_conf, W_bbox, b_bbox):
    B, C, H, W = feat_map.shape

    wall = jnp.zeros((64, C), jnp.float32)
    wall = wall.at[_Q0:_Q0 + 4].set(W_quat)
    wall = wall.at[_T0:_T0 + 3].set(W_trans)
    wall = wall.at[_E0:_E0 + 32].set(W_embed)
    wall = wall.at[_C0:_C0 + 1].set(W_conf)
    wall = wall.at[_B0:_B0 + 4].set(W_bbox)
    ball = jnp.zeros((64, 1), jnp.float32)
    ball = ball.at[_Q0:_Q0 + 4, 0].set(b_quat)
    ball = ball.at[_T0:_T0 + 3, 0].set(b_trans)
    ball = ball.at[_E0:_E0 + 32, 0].set(b_embed)
    ball = ball.at[_C0:_C0 + 1, 0].set(b_conf)
    ball = ball.at[_B0:_B0 + 4, 0].set(b_bbox)

    grid = (B // _BB, H // _HB)
    ch_spec = lambda o: pl.BlockSpec((_BB, o, _HB, 128),
                                     lambda b, i: (b, 0, i, 0))
    outs = pl.pallas_call(
        _head_kernel,
        grid=grid,
        in_specs=[
            ch_spec(C),
            pl.BlockSpec((64, C), lambda b, i: (0, 0)),
            pl.BlockSpec((64, 1), lambda b, i: (0, 0)),
        ],
        out_specs=[ch_spec(4), ch_spec(3), ch_spec(32), ch_spec(1), ch_spec(4),
                   ch_spec(9)],
        out_shape=[
            jax.ShapeDtypeStruct((B, 4, H, W), jnp.float32),
            jax.ShapeDtypeStruct((B, 3, H, W), jnp.float32),
            jax.ShapeDtypeStruct((B, 32, H, W), jnp.float32),
            jax.ShapeDtypeStruct((B, 1, H, W), jnp.float32),
            jax.ShapeDtypeStruct((B, 4, H, W), jnp.float32),
            jax.ShapeDtypeStruct((B, 9, H, W), jnp.float32),
        ],
        compiler_params=pltpu.CompilerParams(
            dimension_semantics=("parallel", "parallel")),
    )(feat_map, wall, ball)

    quat, trans, embed, conf, bbox, r9 = outs
    global_R = jnp.transpose(r9, (0, 2, 3, 1)).reshape(B, H, W, 3, 3)
    return (quat, trans, embed, conf, bbox, global_R, trans)


# speedup vs baseline: 1.2882x; 1.0470x over previous
import functools

import jax
import jax.numpy as jnp
from jax.experimental import pallas as pl
from jax.experimental.pallas import tpu as pltpu

_Q0, _T0, _E0, _C0, _B0 = 0, 8, 16, 48, 56
_HB = 128
_NB = _HB * 128
_BB = 2


def _head_kernel(f_ref, w_ref, b_ref, quat_ref, trans_ref, embed_ref,
                 conf_ref, bbox_ref, r9_ref):
    for bb in range(_BB):
        f = f_ref[bb].astype(jnp.bfloat16)
        out = jax.lax.dot_general(
            w_ref[...], f, (((1,), (0,)), ((), ())),
            preferred_element_type=jnp.float32)
        out = out + b_ref[...].reshape(64, 1, 1)

        q4 = out[_Q0:_Q0 + 4]
        trans_ref[bb] = out[_T0:_T0 + 3]
        embed_ref[bb] = out[_E0:_E0 + 32]
        conf_ref[bb] = jax.nn.sigmoid(out[_C0:_C0 + 1])
        bbox_ref[bb] = out[_B0:_B0 + 4]

        s = jnp.sum(q4 * q4, axis=0, keepdims=True)
        inv = 1.0 / (jnp.sqrt(s) + 1e-8)
        qn = q4 * inv
        quat_ref[bb] = qn

        w = qn[0]
        x = qn[1]
        y = qn[2]
        z = qn[3]
        xx = 2.0 * x * x
        yy = 2.0 * y * y
        zz = 2.0 * z * z
        xy = 2.0 * x * y
        xz = 2.0 * x * z
        yz = 2.0 * y * z
        xw = 2.0 * x * w
        yw = 2.0 * y * w
        zw = 2.0 * z * w
        rs = (1.0 - (yy + zz), xy - zw, xz + yw,
              xy + zw, 1.0 - (xx + zz), yz - xw,
              xz - yw, yz + xw, 1.0 - (xx + yy))
        for k, r in enumerate(rs):
            r9_ref[bb, k] = r


@functools.partial(jax.jit, static_argnames=())
def kernel(feat_map, W_quat, b_quat, W_trans, b_trans, W_embed, b_embed,
           W_conf, b_conf, W_bbox, b_bbox):
    B, C, H, W = feat_map.shape

    wall = jnp.zeros((64, C), jnp.float32)
    wall = wall.at[_Q0:_Q0 + 4].set(W_quat)
    wall = wall.at[_T0:_T0 + 3].set(W_trans)
    wall = wall.at[_E0:_E0 + 32].set(W_embed)
    wall = wall.at[_C0:_C0 + 1].set(W_conf)
    wall = wall.at[_B0:_B0 + 4].set(W_bbox)
    ball = jnp.zeros((64, 1), jnp.float32)
    ball = ball.at[_Q0:_Q0 + 4, 0].set(b_quat)
    ball = ball.at[_T0:_T0 + 3, 0].set(b_trans)
    ball = ball.at[_E0:_E0 + 32, 0].set(b_embed)
    ball = ball.at[_C0:_C0 + 1, 0].set(b_conf)
    ball = ball.at[_B0:_B0 + 4, 0].set(b_bbox)

    grid = (B // _BB, H // _HB)
    ch_spec = lambda o: pl.BlockSpec((_BB, o, _HB, 128),
                                     lambda b, i: (b, 0, i, 0))
    outs = pl.pallas_call(
        _head_kernel,
        grid=grid,
        in_specs=[
            ch_spec(C),
            pl.BlockSpec((64, C), lambda b, i: (0, 0), ),
            pl.BlockSpec((64, 1), lambda b, i: (0, 0)),
        ],
        out_specs=[ch_spec(4), ch_spec(3), ch_spec(32), ch_spec(1), ch_spec(4),
                   ch_spec(9)],
        out_shape=[
            jax.ShapeDtypeStruct((B, 4, H, W), jnp.float32),
            jax.ShapeDtypeStruct((B, 3, H, W), jnp.float32),
            jax.ShapeDtypeStruct((B, 32, H, W), jnp.float32),
            jax.ShapeDtypeStruct((B, 1, H, W), jnp.float32),
            jax.ShapeDtypeStruct((B, 4, H, W), jnp.float32),
            jax.ShapeDtypeStruct((B, 9, H, W), jnp.float32),
        ],
        compiler_params=pltpu.CompilerParams(
            dimension_semantics=("parallel", "parallel")),
    )(feat_map, wall, ball)

    quat, trans, embed, conf, bbox, r9 = outs
    global_R = jnp.transpose(r9, (0, 2, 3, 1)).reshape(B, H, W, 3, 3)
    return (quat, trans, embed, conf, bbox, global_R, trans)
